# baseline (speedup 1.0000x reference)
import functools

import jax
import jax.numpy as jnp
from jax import lax
from jax.experimental import pallas as pl
from jax.experimental.pallas import tpu as pltpu
from jax.experimental.pallas import tpu_sc as plsc

NUM_ITEMS = 100000
NUM_GENRES = 18
EMB = 64
ITEM_D = 5 * EMB
BATCH = 16384

_NC = 2
_NS = 16
_NW = _NC * _NS
_BPW = BATCH // _NW
_C = 128
_NCHUNK = _BPW // _C


def _sc_gather_body(item_idx_hbm, year_idx_hbm, item_table_hbm, year_table_hbm,
                    item_out_hbm, year_out_hbm,
                    idx_i_v, idx_y_v, rows_i_v, rows_y_v, sem_i, sem_y):
    wid = lax.axis_index("s") * _NC + lax.axis_index("c")
    zeros16 = jnp.zeros((16,), jnp.float32)

    def chunk(c, carry):
        base = wid * _BPW + c * _C
        pltpu.sync_copy(item_idx_hbm.at[pl.ds(base, _C)], idx_i_v)
        pltpu.sync_copy(year_idx_hbm.at[pl.ds(base, _C)], idx_y_v)
        cp_i = pltpu.async_copy(item_table_hbm.at[idx_i_v], rows_i_v, sem_i)
        cp_y = pltpu.async_copy(year_table_hbm.at[idx_y_v], rows_y_v, sem_y)
        cp_i.wait()
        cp_y.wait()

        for g in range(_C // 16):
            idxv = idx_i_v[pl.ds(g * 16, 16)]
            nz = 16 - jnp.sum(jnp.minimum(idxv, 1))

            @pl.when(nz > 0)
            def _():
                for r in range(16):
                    s = idxv[r]

                    @pl.when(s == 0)
                    def _():
                        for j in range(ITEM_D // 16):
                            rows_i_v[g * 16 + r, pl.ds(j * 16, 16)] = zeros16

        pltpu.sync_copy(rows_i_v, item_out_hbm.at[pl.ds(base, _C)])
        pltpu.sync_copy(rows_y_v, year_out_hbm.at[pl.ds(base, _C)])
        return carry

    lax.fori_loop(0, _NCHUNK, chunk, 0)


_sc_gather = pl.kernel(
    _sc_gather_body,
    out_type=(
        jax.ShapeDtypeStruct((BATCH, ITEM_D), jnp.float32),
        jax.ShapeDtypeStruct((BATCH, EMB), jnp.float32),
    ),
    mesh=plsc.VectorSubcoreMesh(core_axis_name="c", subcore_axis_name="s"),
    compiler_params=pltpu.CompilerParams(
        use_tc_tiling_on_sc=False, needs_layout_passes=False
    ),
    scratch_types=[
        pltpu.VMEM((_C,), jnp.int32),
        pltpu.VMEM((_C,), jnp.int32),
        pltpu.VMEM((_C, ITEM_D), jnp.float32),
        pltpu.VMEM((_C, EMB), jnp.float32),
        pltpu.SemaphoreType.DMA,
        pltpu.SemaphoreType.DMA,
    ],
)


def _genre_mm_body(g_ref, w_ref, b_ref, o_ref):
    o_ref[...] = (
        jnp.dot(g_ref[...], w_ref[...], preferred_element_type=jnp.float32)
        + b_ref[...]
    )


_GB = 2048


def _genre_matmul(genres, w, b2d):
    return pl.pallas_call(
        _genre_mm_body,
        grid=(BATCH // _GB,),
        in_specs=[
            pl.BlockSpec((_GB, NUM_GENRES), lambda i: (i, 0)),
            pl.BlockSpec((NUM_GENRES, EMB), lambda i: (0, 0)),
            pl.BlockSpec((1, EMB), lambda i: (0, 0)),
        ],
        out_specs=pl.BlockSpec((_GB, EMB), lambda i: (i, 0)),
        out_shape=jax.ShapeDtypeStruct((BATCH, EMB), jnp.float32),
    )(genres, w, b2d)


def kernel(data, item_table, year_table, genre_W, genre_b):
    item_idx = data[:, 0, 0].astype(jnp.int32)
    year_idx = data[:, 0, 1].astype(jnp.int32)
    genres = data[:, 0, 2:]
    genre_out = _genre_matmul(genres, genre_W, genre_b.reshape(1, EMB))
    item_emb, year_emb = _sc_gather(item_idx, year_idx, item_table, year_table)
    return jnp.concatenate([item_emb, year_emb, genre_out], axis=1)


# speedup vs baseline: 3.1093x; 3.1093x over previous
import functools

import jax
import jax.numpy as jnp
from jax import lax
from jax.experimental import pallas as pl
from jax.experimental.pallas import tpu as pltpu
from jax.experimental.pallas import tpu_sc as plsc

NUM_ITEMS = 100000
NUM_GENRES = 18
EMB = 64
ITEM_D = 5 * EMB
BATCH = 16384

_NC = 2
_NS = 16
_NW = _NC * _NS
_BPW = BATCH // _NW
_C = 128
_NCHUNK = _BPW // _C
_G = _C // 16


def _sc_gather_body(item_idx_hbm, year_idx_hbm, item_table_hbm, year_table_hbm,
                    item_out_hbm, year_out_hbm,
                    idx_i_v, idx_y_v, rows_i_v, rows_y_v, sem_i, sem_y):
    wid = lax.axis_index("s") * _NC + lax.axis_index("c")
    zeros16 = jnp.zeros((16,), jnp.float32)

    def chunk(c, carry):
        base = wid * _BPW + c * _C
        pltpu.sync_copy(item_idx_hbm.at[pl.ds(base, _C)], idx_i_v)
        pltpu.sync_copy(year_idx_hbm.at[pl.ds(base, _C)], idx_y_v)

        def issue(g, carry2):
            iv = idx_i_v[pl.ds(g * 16, 16)]
            yv = idx_y_v[pl.ds(g * 16, 16)]
            for r in range(16):
                si = iv[r]
                sy = yv[r]
                pltpu.async_copy(
                    item_table_hbm.at[pl.ds(si, 1)],
                    rows_i_v.at[pl.ds(g * 16 + r, 1)], sem_i)
                pltpu.async_copy(
                    year_table_hbm.at[pl.ds(sy, 1)],
                    rows_y_v.at[pl.ds(g * 16 + r, 1)], sem_y)
            return carry2

        lax.fori_loop(0, _G, issue, 0)

        def drain(g, carry2):
            for r in range(16):
                pltpu.make_async_copy(
                    item_table_hbm.at[pl.ds(0, 1)],
                    rows_i_v.at[pl.ds(0, 1)], sem_i).wait()
                pltpu.make_async_copy(
                    year_table_hbm.at[pl.ds(0, 1)],
                    rows_y_v.at[pl.ds(0, 1)], sem_y).wait()
            return carry2

        lax.fori_loop(0, _G, drain, 0)

        for g in range(_G):
            idxv = idx_i_v[pl.ds(g * 16, 16)]
            nz = 16 - jnp.sum(jnp.minimum(idxv, 1))

            @pl.when(nz > 0)
            def _():
                for r in range(16):
                    s = idxv[r]

                    @pl.when(s == 0)
                    def _():
                        for j in range(ITEM_D // 16):
                            rows_i_v[g * 16 + r, pl.ds(j * 16, 16)] = zeros16

        pltpu.sync_copy(rows_i_v, item_out_hbm.at[pl.ds(base, _C)])
        pltpu.sync_copy(rows_y_v, year_out_hbm.at[pl.ds(base, _C)])
        return carry

    lax.fori_loop(0, _NCHUNK, chunk, 0)


_sc_gather = pl.kernel(
    _sc_gather_body,
    out_type=(
        jax.ShapeDtypeStruct((BATCH, ITEM_D), jnp.float32),
        jax.ShapeDtypeStruct((BATCH, EMB), jnp.float32),
    ),
    mesh=plsc.VectorSubcoreMesh(core_axis_name="c", subcore_axis_name="s"),
    compiler_params=pltpu.CompilerParams(needs_layout_passes=False),
    scratch_types=[
        pltpu.VMEM((_C,), jnp.int32),
        pltpu.VMEM((_C,), jnp.int32),
        pltpu.VMEM((_C, ITEM_D), jnp.float32),
        pltpu.VMEM((_C, EMB), jnp.float32),
        pltpu.SemaphoreType.DMA,
        pltpu.SemaphoreType.DMA,
    ],
)


def _genre_mm_body(g_ref, w_ref, b_ref, o_ref):
    o_ref[...] = (
        jnp.dot(g_ref[...], w_ref[...], preferred_element_type=jnp.float32)
        + b_ref[...]
    )


_GB = 2048


def _genre_matmul(genres, w, b2d):
    return pl.pallas_call(
        _genre_mm_body,
        grid=(BATCH // _GB,),
        in_specs=[
            pl.BlockSpec((_GB, NUM_GENRES), lambda i: (i, 0)),
            pl.BlockSpec((NUM_GENRES, EMB), lambda i: (0, 0)),
            pl.BlockSpec((1, EMB), lambda i: (0, 0)),
        ],
        out_specs=pl.BlockSpec((_GB, EMB), lambda i: (i, 0)),
        out_shape=jax.ShapeDtypeStruct((BATCH, EMB), jnp.float32),
    )(genres, w, b2d)


def kernel(data, item_table, year_table, genre_W, genre_b):
    item_idx = data[:, 0, 0].astype(jnp.int32)
    year_idx = data[:, 0, 1].astype(jnp.int32)
    genres = data[:, 0, 2:]
    genre_out = _genre_matmul(genres, genre_W, genre_b.reshape(1, EMB))
    item_emb, year_emb = _sc_gather(item_idx, year_idx, item_table, year_table)
    return jnp.concatenate([item_emb, year_emb, genre_out], axis=1)


# speedup vs baseline: 3.5108x; 1.1291x over previous
import functools

import jax
import jax.numpy as jnp
from jax import lax
from jax.experimental import pallas as pl
from jax.experimental.pallas import tpu as pltpu
from jax.experimental.pallas import tpu_sc as plsc

NUM_ITEMS = 100000
NUM_GENRES = 18
EMB = 64
ITEM_D = 5 * EMB
COMB_D = ITEM_D + EMB
OUT_D = COMB_D + EMB
BATCH = 16384

_NC = 2
_NS = 16
_NW = _NC * _NS
_BPW = BATCH // _NW
_C = 128
_NCHUNK = _BPW // _C
_G = _C // 16


def _sc_gather_body(item_idx_hbm, year_idx_hbm, item_table_hbm, year_table_hbm,
                    item_out_hbm, year_out_hbm,
                    idx_i_v, idx_y_v, rows_i_v, rows_y_v, sem_i, sem_y):
    wid = lax.axis_index("s") * _NC + lax.axis_index("c")
    zeros16 = jnp.zeros((16,), jnp.float32)

    def chunk(c, carry):
        base = wid * _BPW + c * _C
        pltpu.sync_copy(item_idx_hbm.at[pl.ds(base, _C)], idx_i_v)
        pltpu.sync_copy(year_idx_hbm.at[pl.ds(base, _C)], idx_y_v)

        def issue(g, carry2):
            iv = idx_i_v[pl.ds(g * 16, 16)]
            yv = idx_y_v[pl.ds(g * 16, 16)]
            for r in range(16):
                si = iv[r]
                sy = yv[r]
                pltpu.async_copy(
                    item_table_hbm.at[pl.ds(si, 1)],
                    rows_i_v.at[pl.ds(g * 16 + r, 1)], sem_i)
                pltpu.async_copy(
                    year_table_hbm.at[pl.ds(sy, 1)],
                    rows_y_v.at[pl.ds(g * 16 + r, 1)], sem_y)
            return carry2

        lax.fori_loop(0, _G, issue, 0)

        def drain(g, carry2):
            for r in range(16):
                pltpu.make_async_copy(
                    item_table_hbm.at[pl.ds(0, 1)],
                    rows_i_v.at[pl.ds(0, 1)], sem_i).wait()
                pltpu.make_async_copy(
                    year_table_hbm.at[pl.ds(0, 1)],
                    rows_y_v.at[pl.ds(0, 1)], sem_y).wait()
            return carry2

        lax.fori_loop(0, _G, drain, 0)

        for g in range(_G):
            idxv = idx_i_v[pl.ds(g * 16, 16)]
            nz = 16 - jnp.sum(jnp.minimum(idxv, 1))

            @pl.when(nz > 0)
            def _():
                for r in range(16):
                    s = idxv[r]

                    @pl.when(s == 0)
                    def _():
                        for j in range(ITEM_D // 16):
                            rows_i_v[g * 16 + r, pl.ds(j * 16, 16)] = zeros16

        pltpu.sync_copy(rows_i_v, item_out_hbm.at[pl.ds(base, _C)])
        pltpu.sync_copy(rows_y_v, year_out_hbm.at[pl.ds(base, _C)])
        return carry

    lax.fori_loop(0, _NCHUNK, chunk, 0)


_sc_gather = pl.kernel(
    _sc_gather_body,
    out_type=(
        jax.ShapeDtypeStruct((BATCH, ITEM_D), jnp.float32),
        jax.ShapeDtypeStruct((BATCH, EMB), jnp.float32),
    ),
    mesh=plsc.VectorSubcoreMesh(core_axis_name="c", subcore_axis_name="s"),
    compiler_params=pltpu.CompilerParams(needs_layout_passes=False),
    scratch_types=[
        pltpu.VMEM((_C,), jnp.int32),
        pltpu.VMEM((_C,), jnp.int32),
        pltpu.VMEM((_C, ITEM_D), jnp.float32),
        pltpu.VMEM((_C, EMB), jnp.float32),
        pltpu.SemaphoreType.DMA,
        pltpu.SemaphoreType.DMA,
    ],
)


_TB = 2048


def _assemble_body(item_ref, year_ref, genres_t_ref, w_ref, b_ref, out_ref):
    out_ref[pl.ds(0, ITEM_D), :] = item_ref[...].T
    out_ref[pl.ds(ITEM_D, EMB), :] = year_ref[...].T
    go_t = jnp.dot(w_ref[...].T, genres_t_ref[...],
                   preferred_element_type=jnp.float32)
    out_ref[pl.ds(COMB_D, EMB), :] = go_t + b_ref[...].T


def _assemble(item_emb, year_emb, genres_t, w, b2d):
    return pl.pallas_call(
        _assemble_body,
        grid=(BATCH // _TB,),
        in_specs=[
            pl.BlockSpec((_TB, ITEM_D), lambda i: (i, 0)),
            pl.BlockSpec((_TB, EMB), lambda i: (i, 0)),
            pl.BlockSpec((NUM_GENRES, _TB), lambda i: (0, i)),
            pl.BlockSpec((NUM_GENRES, EMB), lambda i: (0, 0)),
            pl.BlockSpec((1, EMB), lambda i: (0, 0)),
        ],
        out_specs=pl.BlockSpec((OUT_D, _TB), lambda i: (0, i)),
        out_shape=jax.ShapeDtypeStruct((OUT_D, BATCH), jnp.float32),
    )(item_emb, year_emb, genres_t, w, b2d)


def kernel(data, item_table, year_table, genre_W, genre_b):
    item_idx = data[:, 0, 0].astype(jnp.int32)
    year_idx = data[:, 0, 1].astype(jnp.int32)
    genres_t = data[:, 0, 2:].T
    item_emb, year_emb = _sc_gather(item_idx, year_idx, item_table, year_table)
    out_t = _assemble(item_emb, year_emb, genres_t, genre_W,
                      genre_b.reshape(1, EMB))
    return out_t.T


# speedup vs baseline: 4.4149x; 1.2575x over previous
import functools

import jax
import jax.numpy as jnp
from jax import lax
from jax.experimental import pallas as pl
from jax.experimental.pallas import tpu as pltpu
from jax.experimental.pallas import tpu_sc as plsc

NUM_ITEMS = 100000
NUM_GENRES = 18
EMB = 64
ITEM_D = 5 * EMB
COMB_D = ITEM_D + EMB
OUT_D = COMB_D + EMB
BATCH = 16384

_NC = 2
_NS = 16
_NW = _NC * _NS
_BPW = BATCH // _NW
_C = 128
_NCHUNK = _BPW // _C
_G = _C // 16


def _conv_body(src_ref, dst_ref):
    dst_ref[...] = src_ref[...].T


def _convert_table(table_t, n_rows, d, row_blk):
    n_blk = (n_rows + row_blk - 1) // row_blk
    return pl.pallas_call(
        _conv_body,
        grid=(n_blk,),
        in_specs=[pl.BlockSpec((d, row_blk), lambda i: (0, i))],
        out_specs=pl.BlockSpec((row_blk, d), lambda i: (i, 0)),
        out_shape=jax.ShapeDtypeStruct((n_rows, d), jnp.float32),
    )(table_t)


def _sc_gather_body(item_idx_hbm, year_idx_hbm, item_table_hbm, year_table_hbm,
                    item_out_hbm, year_out_hbm,
                    idx_i_v, idx_y_v, rows_i_v, rows_y_v, sem_i, sem_y):
    wid = lax.axis_index("s") * _NC + lax.axis_index("c")

    def chunk(c, carry):
        base = wid * _BPW + c * _C
        pltpu.sync_copy(item_idx_hbm.at[pl.ds(base, _C)], idx_i_v)
        pltpu.sync_copy(year_idx_hbm.at[pl.ds(base, _C)], idx_y_v)

        def issue(g, carry2):
            iv = idx_i_v[pl.ds(g * 16, 16)]
            yv = idx_y_v[pl.ds(g * 16, 16)]
            for r in range(16):
                si = iv[r]
                sy = yv[r]
                pltpu.async_copy(
                    item_table_hbm.at[pl.ds(si, 1)],
                    rows_i_v.at[pl.ds(g * 16 + r, 1)], sem_i)
                pltpu.async_copy(
                    year_table_hbm.at[pl.ds(sy, 1)],
                    rows_y_v.at[pl.ds(g * 16 + r, 1)], sem_y)
            return carry2

        lax.fori_loop(0, _G, issue, 0)

        def drain(g, carry2):
            for r in range(16):
                pltpu.make_async_copy(
                    item_table_hbm.at[pl.ds(0, 1)],
                    rows_i_v.at[pl.ds(0, 1)], sem_i).wait()
                pltpu.make_async_copy(
                    year_table_hbm.at[pl.ds(0, 1)],
                    rows_y_v.at[pl.ds(0, 1)], sem_y).wait()
            return carry2

        lax.fori_loop(0, _G, drain, 0)

        pltpu.sync_copy(rows_i_v, item_out_hbm.at[pl.ds(base, _C)])
        pltpu.sync_copy(rows_y_v, year_out_hbm.at[pl.ds(base, _C)])
        return carry

    lax.fori_loop(0, _NCHUNK, chunk, 0)


_sc_gather = pl.kernel(
    _sc_gather_body,
    out_type=(
        jax.ShapeDtypeStruct((BATCH, ITEM_D), jnp.float32),
        jax.ShapeDtypeStruct((BATCH, EMB), jnp.float32),
    ),
    mesh=plsc.VectorSubcoreMesh(core_axis_name="c", subcore_axis_name="s"),
    compiler_params=pltpu.CompilerParams(needs_layout_passes=False),
    scratch_types=[
        pltpu.VMEM((_C,), jnp.int32),
        pltpu.VMEM((_C,), jnp.int32),
        pltpu.VMEM((_C, ITEM_D), jnp.float32),
        pltpu.VMEM((_C, EMB), jnp.float32),
        pltpu.SemaphoreType.DMA,
        pltpu.SemaphoreType.DMA,
    ],
)


_TB = 2048


def _assemble_body(item_ref, year_ref, idxf_ref, genres_t_ref, w_ref, b_ref,
                   out_ref):
    sel = (idxf_ref[0] != 0.0).astype(jnp.float32)
    it_t = item_ref[...].astype(jnp.float32).T
    out_ref[pl.ds(0, ITEM_D), :] = it_t * sel
    out_ref[pl.ds(ITEM_D, EMB), :] = year_ref[...].astype(jnp.float32).T
    go_t = jnp.dot(w_ref[...].T, genres_t_ref[...],
                   preferred_element_type=jnp.float32)
    out_ref[pl.ds(COMB_D, EMB), :] = go_t + b_ref[...].T


def _assemble(item_emb, year_emb, idxf3, genres_t, w, b2d):
    return pl.pallas_call(
        _assemble_body,
        grid=(BATCH // _TB,),
        in_specs=[
            pl.BlockSpec((_TB, ITEM_D), lambda i: (i, 0)),
            pl.BlockSpec((_TB, EMB), lambda i: (i, 0)),
            pl.BlockSpec((1, 1, _TB), lambda i: (i, 0, 0)),
            pl.BlockSpec((NUM_GENRES, _TB), lambda i: (0, i)),
            pl.BlockSpec((NUM_GENRES, EMB), lambda i: (0, 0)),
            pl.BlockSpec((1, EMB), lambda i: (0, 0)),
        ],
        out_specs=pl.BlockSpec((OUT_D, _TB), lambda i: (0, i)),
        out_shape=jax.ShapeDtypeStruct((OUT_D, BATCH), jnp.float32),
    )(item_emb, year_emb, idxf3, genres_t, w, b2d)


def kernel(data, item_table, year_table, genre_W, genre_b):
    item_idx = data[:, 0, 0].astype(jnp.int32)
    year_idx = data[:, 0, 1].astype(jnp.int32)
    idxf3 = data[:, 0, 0].reshape(BATCH // _TB, 1, _TB)
    genres_t = data[:, 0, 2:].T

    item_bf = _convert_table(item_table.T, NUM_ITEMS + 1, ITEM_D, 4096)
    year_bf = _convert_table(year_table.T, NUM_ITEMS, EMB, 8192)

    item_emb, year_emb = _sc_gather(item_idx, year_idx, item_bf, year_bf)
    out_t = _assemble(item_emb, year_emb, idxf3, genres_t, genre_W,
                      genre_b.reshape(1, EMB))
    return out_t.T


# speedup vs baseline: 4.4854x; 1.0160x over previous
import functools

import jax
import jax.numpy as jnp
from jax import lax
from jax.experimental import pallas as pl
from jax.experimental.pallas import tpu as pltpu
from jax.experimental.pallas import tpu_sc as plsc

NUM_ITEMS = 100000
NUM_GENRES = 18
EMB = 64
ITEM_D = 5 * EMB
COMB_D = ITEM_D + EMB
OUT_D = COMB_D + EMB
BATCH = 16384

_NC = 2
_NS = 16
_NW = _NC * _NS
_BPW = BATCH // _NW
_C = 128
_NCHUNK = _BPW // _C
_G = _C // 16


def _conv_body(src_ref, dst_ref):
    xt = src_ref[...].T
    u = jax.lax.bitcast_convert_type(xt, jnp.uint32)
    r = u + jnp.uint32(0x7FFF) + ((u >> jnp.uint32(16)) & jnp.uint32(1))
    h = xt.shape[1] // 2
    hi = r[:, :h] & jnp.uint32(0xFFFF0000)
    lo = r[:, h:] >> jnp.uint32(16)
    dst_ref[...] = jax.lax.bitcast_convert_type(hi | lo, jnp.float32)


def _convert_table(table_t, n_rows, d, row_blk):
    n_blk = (n_rows + row_blk - 1) // row_blk
    return pl.pallas_call(
        _conv_body,
        grid=(n_blk,),
        in_specs=[pl.BlockSpec((d, row_blk), lambda i: (0, i))],
        out_specs=pl.BlockSpec((row_blk, d // 2), lambda i: (i, 0)),
        out_shape=jax.ShapeDtypeStruct((n_rows, d // 2), jnp.float32),
    )(table_t)


def _unpack_t(packed):
    u = jax.lax.bitcast_convert_type(packed, jnp.uint32)
    hi = jax.lax.bitcast_convert_type(u & jnp.uint32(0xFFFF0000), jnp.float32)
    lo = jax.lax.bitcast_convert_type(u << jnp.uint32(16), jnp.float32)
    return hi.T, lo.T


def _sc_gather_body(item_idx_hbm, year_idx_hbm, item_table_hbm, year_table_hbm,
                    item_out_hbm, year_out_hbm,
                    idx_i_v, idx_y_v, rows_i_v, rows_y_v, sem_i, sem_y):
    wid = lax.axis_index("s") * _NC + lax.axis_index("c")

    def chunk(c, carry):
        base = wid * _BPW + c * _C
        pltpu.sync_copy(item_idx_hbm.at[pl.ds(base, _C)], idx_i_v)
        pltpu.sync_copy(year_idx_hbm.at[pl.ds(base, _C)], idx_y_v)

        def issue(g, carry2):
            iv = idx_i_v[pl.ds(g * 16, 16)]
            yv = idx_y_v[pl.ds(g * 16, 16)]
            for r in range(16):
                si = iv[r]
                sy = yv[r]
                pltpu.async_copy(
                    item_table_hbm.at[pl.ds(si, 1)],
                    rows_i_v.at[pl.ds(g * 16 + r, 1)], sem_i)
                pltpu.async_copy(
                    year_table_hbm.at[pl.ds(sy, 1)],
                    rows_y_v.at[pl.ds(g * 16 + r, 1)], sem_y)
            return carry2

        lax.fori_loop(0, _G, issue, 0)

        def drain(g, carry2):
            for r in range(16):
                pltpu.make_async_copy(
                    item_table_hbm.at[pl.ds(0, 1)],
                    rows_i_v.at[pl.ds(0, 1)], sem_i).wait()
                pltpu.make_async_copy(
                    year_table_hbm.at[pl.ds(0, 1)],
                    rows_y_v.at[pl.ds(0, 1)], sem_y).wait()
            return carry2

        lax.fori_loop(0, _G, drain, 0)

        pltpu.sync_copy(rows_i_v, item_out_hbm.at[pl.ds(base, _C)])
        pltpu.sync_copy(rows_y_v, year_out_hbm.at[pl.ds(base, _C)])
        return carry

    lax.fori_loop(0, _NCHUNK, chunk, 0)


_sc_gather = pl.kernel(
    _sc_gather_body,
    out_type=(
        jax.ShapeDtypeStruct((BATCH, ITEM_D // 2), jnp.float32),
        jax.ShapeDtypeStruct((BATCH, EMB // 2), jnp.float32),
    ),
    mesh=plsc.VectorSubcoreMesh(core_axis_name="c", subcore_axis_name="s"),
    compiler_params=pltpu.CompilerParams(needs_layout_passes=False),
    scratch_types=[
        pltpu.VMEM((_C,), jnp.int32),
        pltpu.VMEM((_C,), jnp.int32),
        pltpu.VMEM((_C, ITEM_D // 2), jnp.float32),
        pltpu.VMEM((_C, EMB // 2), jnp.float32),
        pltpu.SemaphoreType.DMA,
        pltpu.SemaphoreType.DMA,
    ],
)


_TB = 2048


def _assemble_body(item_ref, year_ref, idxf_ref, genres_t_ref, w_ref, b_ref,
                   out_ref):
    sel = (idxf_ref[0] != 0.0).astype(jnp.float32)
    it_hi, it_lo = _unpack_t(item_ref[...])
    out_ref[pl.ds(0, ITEM_D // 2), :] = it_hi * sel
    out_ref[pl.ds(ITEM_D // 2, ITEM_D // 2), :] = it_lo * sel
    yr_hi, yr_lo = _unpack_t(year_ref[...])
    out_ref[pl.ds(ITEM_D, EMB // 2), :] = yr_hi
    out_ref[pl.ds(ITEM_D + EMB // 2, EMB // 2), :] = yr_lo
    go_t = jnp.dot(w_ref[...].T, genres_t_ref[...],
                   preferred_element_type=jnp.float32)
    out_ref[pl.ds(COMB_D, EMB), :] = go_t + b_ref[...].T


def _assemble(item_emb, year_emb, idxf3, genres_t, w, b2d):
    return pl.pallas_call(
        _assemble_body,
        grid=(BATCH // _TB,),
        in_specs=[
            pl.BlockSpec((_TB, ITEM_D // 2), lambda i: (i, 0)),
            pl.BlockSpec((_TB, EMB // 2), lambda i: (i, 0)),
            pl.BlockSpec((1, 1, _TB), lambda i: (i, 0, 0)),
            pl.BlockSpec((NUM_GENRES, _TB), lambda i: (0, i)),
            pl.BlockSpec((NUM_GENRES, EMB), lambda i: (0, 0)),
            pl.BlockSpec((1, EMB), lambda i: (0, 0)),
        ],
        out_specs=pl.BlockSpec((OUT_D, _TB), lambda i: (0, i)),
        out_shape=jax.ShapeDtypeStruct((OUT_D, BATCH), jnp.float32),
    )(item_emb, year_emb, idxf3, genres_t, w, b2d)


def kernel(data, item_table, year_table, genre_W, genre_b):
    item_idx = data[:, 0, 0].astype(jnp.int32)
    year_idx = data[:, 0, 1].astype(jnp.int32)
    idxf3 = data[:, 0, 0].reshape(BATCH // _TB, 1, _TB)
    genres_t = data[:, 0, 2:].T

    item_bf = _convert_table(item_table.T, NUM_ITEMS + 1, ITEM_D, 4096)
    year_bf = _convert_table(year_table.T, NUM_ITEMS, EMB, 8192)

    item_emb, year_emb = _sc_gather(item_idx, year_idx, item_bf, year_bf)
    out_t = _assemble(item_emb, year_emb, idxf3, genres_t, genre_W,
                      genre_b.reshape(1, EMB))
    return out_t.T


# speedup vs baseline: 5.4161x; 1.2075x over previous
import functools

import jax
import jax.numpy as jnp
from jax import lax
from jax.experimental import pallas as pl
from jax.experimental.pallas import tpu as pltpu
from jax.experimental.pallas import tpu_sc as plsc

NUM_ITEMS = 100000
NUM_GENRES = 18
EMB = 64
ITEM_D = 5 * EMB
ITEM_P = ITEM_D // 2
YEAR_P = EMB // 2
COMB_D = ITEM_D + EMB
OUT_D = COMB_D + EMB
BATCH = 16384

_NC = 2
_NS = 16
_NW = _NC * _NS
_BPW = BATCH // _NW
_G = _BPW // 16


def _conv_body(src_ref, dst_ref):
    x = src_ref[...]
    u = jax.lax.bitcast_convert_type(x, jnp.uint32)
    r = u + jnp.uint32(0x7FFF) + ((u >> jnp.uint32(16)) & jnp.uint32(1))
    h = x.shape[0] // 2
    hi = r[:h, :] & jnp.uint32(0xFFFF0000)
    lo = r[h:, :] >> jnp.uint32(16)
    packed = jax.lax.bitcast_convert_type(hi | lo, jnp.float32)
    dst_ref[...] = packed.T


def _convert_table(table_t, n_rows, d, row_blk):
    n_blk = (n_rows + row_blk - 1) // row_blk
    return pl.pallas_call(
        _conv_body,
        grid=(n_blk,),
        in_specs=[pl.BlockSpec((d, row_blk), lambda i: (0, i))],
        out_specs=pl.BlockSpec((row_blk, d // 2), lambda i: (i, 0)),
        out_shape=jax.ShapeDtypeStruct((n_rows, d // 2), jnp.float32),
    )(table_t)


def _make_sc_gather(width, chunk_rows):
    n_chunk = _BPW // chunk_rows
    n_grp = chunk_rows // 16

    def body(idx_hbm, table_hbm, out_hbm, idx_v, rows_v, sem):
        wid = lax.axis_index("s") * _NC + lax.axis_index("c")

        def chunk(c, carry):
            base = wid * _BPW + c * chunk_rows
            pltpu.sync_copy(idx_hbm.at[pl.ds(base, chunk_rows)], idx_v)

            def issue(g, carry2):
                iv = idx_v[pl.ds(g * 16, 16)]
                for r in range(16):
                    pltpu.async_copy(
                        table_hbm.at[pl.ds(iv[r], 1)],
                        rows_v.at[pl.ds(g * 16 + r, 1)], sem)
                return carry2

            lax.fori_loop(0, n_grp, issue, 0)

            def drain(g, carry2):
                for r in range(16):
                    pltpu.make_async_copy(
                        table_hbm.at[pl.ds(0, 1)],
                        rows_v.at[pl.ds(0, 1)], sem).wait()
                return carry2

            lax.fori_loop(0, n_grp, drain, 0)
            pltpu.sync_copy(rows_v, out_hbm.at[pl.ds(base, chunk_rows)])
            return carry

        lax.fori_loop(0, n_chunk, chunk, 0)

    return pl.kernel(
        body,
        out_type=jax.ShapeDtypeStruct((BATCH, width), jnp.float32),
        mesh=plsc.VectorSubcoreMesh(core_axis_name="c", subcore_axis_name="s"),
        compiler_params=pltpu.CompilerParams(needs_layout_passes=False),
        scratch_types=[
            pltpu.VMEM((chunk_rows,), jnp.int32),
            pltpu.VMEM((chunk_rows, width), jnp.float32),
            pltpu.SemaphoreType.DMA,
        ],
    )


_sc_gather_item = _make_sc_gather(ITEM_P, 256)
_sc_gather_year = _make_sc_gather(YEAR_P, 512)


_TB = 2048


def _unpack_t(packed):
    u = jax.lax.bitcast_convert_type(packed, jnp.uint32)
    hi = jax.lax.bitcast_convert_type(u & jnp.uint32(0xFFFF0000), jnp.float32)
    lo = jax.lax.bitcast_convert_type(u << jnp.uint32(16), jnp.float32)
    return hi.T, lo.T


def _assemble_body(item_ref, year_ref, idxf_ref, genres_t_ref, w_ref, b_ref,
                   out_ref):
    sel = (idxf_ref[0] != 0.0).astype(jnp.float32)
    it_hi, it_lo = _unpack_t(item_ref[...])
    out_ref[pl.ds(0, ITEM_P), :] = it_hi * sel
    out_ref[pl.ds(ITEM_P, ITEM_P), :] = it_lo * sel
    yr_hi, yr_lo = _unpack_t(year_ref[...])
    out_ref[pl.ds(ITEM_D, YEAR_P), :] = yr_hi
    out_ref[pl.ds(ITEM_D + YEAR_P, YEAR_P), :] = yr_lo
    go_t = jnp.dot(w_ref[...].T, genres_t_ref[...],
                   preferred_element_type=jnp.float32)
    out_ref[pl.ds(COMB_D, EMB), :] = go_t + b_ref[...].T


def _assemble(item_emb, year_emb, idxf3, genres_t, w, b2d):
    return pl.pallas_call(
        _assemble_body,
        grid=(BATCH // _TB,),
        in_specs=[
            pl.BlockSpec((_TB, ITEM_P), lambda i: (i, 0)),
            pl.BlockSpec((_TB, YEAR_P), lambda i: (i, 0)),
            pl.BlockSpec((1, 1, _TB), lambda i: (i, 0, 0)),
            pl.BlockSpec((NUM_GENRES, _TB), lambda i: (0, i)),
            pl.BlockSpec((NUM_GENRES, EMB), lambda i: (0, 0)),
            pl.BlockSpec((1, EMB), lambda i: (0, 0)),
        ],
        out_specs=pl.BlockSpec((OUT_D, _TB), lambda i: (0, i)),
        out_shape=jax.ShapeDtypeStruct((OUT_D, BATCH), jnp.float32),
    )(item_emb, year_emb, idxf3, genres_t, w, b2d)


def kernel(data, item_table, year_table, genre_W, genre_b):
    item_idx = data[:, 0, 0].astype(jnp.int32)
    year_idx = data[:, 0, 1].astype(jnp.int32)
    idxf3 = data[:, 0, 0].reshape(BATCH // _TB, 1, _TB)
    genres_t = data[:, 0, 2:].T

    year_bf = _convert_table(year_table.T, NUM_ITEMS, EMB, 16384)
    year_emb = _sc_gather_year(year_idx, year_bf)
    item_bf = _convert_table(item_table.T, NUM_ITEMS + 1, ITEM_D, 8192)
    item_emb = _sc_gather_item(item_idx, item_bf)

    out_t = _assemble(item_emb, year_emb, idxf3, genres_t, genre_W,
                      genre_b.reshape(1, EMB))
    return out_t.T
